# Initial kernel scaffold; baseline (speedup 1.0000x reference)
import jax
import jax.numpy as jnp
from jax.experimental import pallas as pl


def kernel(x, batch, ei, W):
    raise NotImplementedError("write your pallas kernel here")


# speedup vs baseline: 20.3885x; 20.3885x over previous
import functools

import jax
import jax.numpy as jnp
import numpy as np
from jax import lax
from jax.experimental import pallas as pl
from jax.experimental.pallas import tpu as pltpu
from jax.experimental.pallas import tpu_sc as plsc

Fin = 128
Fqk = 128
Fv = 128
H = 16
Fh = Fqk // H
N = 10000
E = 320000

NW = 32
NP = 10240
BN = NP // NW
SUB = 2
HB = BN // SUB
CH = 128
EPAD = E + 4 * CH
RSP = NP + 16


def _proj_body(x_ref, w_ref, o_ref):
    o_ref[...] = lax.dot_general(
        x_ref[...], w_ref[...],
        dimension_numbers=(((1,), (1,)), ((), ())),
        preferred_element_type=jnp.float32,
    )


def _sc_body(qT_h, kT_h, vT_h, src_h, dst_h, rs_h, out_h,
             q_blk, acc, esum, k_ch, v_ch, s_ch, d_ch, rs_v, sem):
    wid = lax.axis_index("s") * 2 + lax.axis_index("c")
    n0 = wid * BN

    pltpu.sync_copy(rs_h.at[pl.ds(n0, BN + 16)], rs_v)

    for sb in range(SUB):
        nb = n0 + sb * HB
        pltpu.sync_copy(qT_h.at[pl.ds(nb, HB)], q_blk)

        def _zero(ln, _):
            z = jnp.zeros((16,), jnp.float32)
            esum[ln] = z
            for f in range(Fh):
                acc[ln, pl.ds(f * H, H)] = z
            return _

        lax.fori_loop(0, HB, _zero, None)

        e_lo = rs_v[pl.ds(sb * HB, 16)][0]
        e_hi = rs_v[pl.ds((sb + 1) * HB, 16)][0]
        e_base = (e_lo // 8) * 8
        nch = (e_hi - e_base + CH - 1) // CH

        def _chunk(c, _):
            base = e_base + c * CH
            pltpu.sync_copy(src_h.at[pl.ds(base, CH + 16)], s_ch)
            pltpu.sync_copy(dst_h.at[pl.ds(base, CH)], d_ch)
            pltpu.async_copy(kT_h.at[d_ch], k_ch, sem).wait()
            pltpu.async_copy(vT_h.at[d_ch], v_ch, sem).wait()

            j0 = jnp.maximum(e_lo - base, 0)
            j1 = jnp.minimum(e_hi - base, CH)

            def _edge(j, __):
                ln = s_ch[pl.ds(j, 16)][0] - nb
                aw = q_blk[ln, pl.ds(0, H)] * k_ch[j, pl.ds(0, H)]
                for f in range(1, Fh):
                    aw = aw + (q_blk[ln, pl.ds(f * H, H)]
                               * k_ch[j, pl.ds(f * H, H)])
                ex = jnp.exp(aw)
                esum[ln] = esum[ln] + ex
                for f in range(Fh):
                    acc[ln, pl.ds(f * H, H)] = (
                        acc[ln, pl.ds(f * H, H)] + ex * v_ch[j, pl.ds(f * H, H)])
                return __

            lax.fori_loop(j0, j1, _edge, None)
            return _

        lax.fori_loop(0, nch, _chunk, None)

        def _norm(ln, _):
            es = esum[ln]
            inv = jnp.where(es > 0.0, 1.0 / es, 0.0)
            for f in range(Fh):
                acc[ln, pl.ds(f * H, H)] = acc[ln, pl.ds(f * H, H)] * inv
            return _

        lax.fori_loop(0, HB, _norm, None)
        pltpu.sync_copy(acc, out_h.at[pl.ds(nb, HB)])


@functools.lru_cache(maxsize=None)
def _sc_attn():
    return pl.kernel(
        _sc_body,
        out_type=jax.ShapeDtypeStruct((NP, Fqk), jnp.float32),
        mesh=plsc.VectorSubcoreMesh(core_axis_name="c", subcore_axis_name="s"),
        scratch_types=[
            pltpu.VMEM((HB, Fqk), jnp.float32),
            pltpu.VMEM((HB, Fqk), jnp.float32),
            pltpu.VMEM((HB, H), jnp.float32),
            pltpu.VMEM((CH, Fqk), jnp.float32),
            pltpu.VMEM((CH, Fqk), jnp.float32),
            pltpu.VMEM((CH + 16,), jnp.int32),
            pltpu.VMEM((CH,), jnp.int32),
            pltpu.VMEM((BN + 16,), jnp.int32),
            pltpu.SemaphoreType.DMA,
        ],
    )

_PERM = np.array([[h * Fh + f for h in range(H)] for f in range(Fh)]).reshape(-1)


def kernel(x, batch, ei, W):
    del batch
    src = ei[0].astype(jnp.int32)
    dst = ei[1].astype(jnp.int32)

    scaling = float(Fh) ** (-0.5)
    Wq = W[:Fqk][_PERM] * scaling
    Wk = W[Fqk:2 * Fqk][_PERM]
    Wv = W[2 * Fqk:][_PERM]
    Wp = jnp.concatenate([Wq, Wk, Wv], axis=0)

    x_pad = jnp.zeros((NP, Fin), jnp.float32).at[:N].set(x)

    RB = 512
    proj = pl.pallas_call(
        _proj_body,
        grid=(NP // RB,),
        in_specs=[
            pl.BlockSpec((RB, Fin), lambda i: (i, 0)),
            pl.BlockSpec((3 * Fqk, Fin), lambda i: (0, 0)),
        ],
        out_specs=pl.BlockSpec((RB, 3 * Fqk), lambda i: (i, 0)),
        out_shape=jax.ShapeDtypeStruct((NP, 3 * Fqk), jnp.float32),
    )(x_pad, Wp)

    qT = proj[:, :Fqk]
    kT = proj[:, Fqk:2 * Fqk]
    vT = proj[:, 2 * Fqk:]

    row_start = jnp.searchsorted(src, jnp.arange(NP + 1, dtype=jnp.int32),
                                 side="left").astype(jnp.int32)
    rs_pad = jnp.concatenate(
        [row_start, jnp.full((RSP - NP - 1,), E, jnp.int32)])
    src_p = jnp.concatenate([src, jnp.zeros((EPAD - E,), jnp.int32)])
    dst_p = jnp.concatenate([dst, jnp.zeros((EPAD - E,), jnp.int32)])

    out = _sc_attn()(qT, kT, vT, src_p, dst_p, rs_pad)
    return out[:N].reshape(N, Fh, H).transpose(0, 2, 1).reshape(N, H * Fh)


# speedup vs baseline: 191.7206x; 9.4034x over previous
import functools

import jax
import jax.numpy as jnp
import numpy as np
from jax import lax
from jax.experimental import pallas as pl
from jax.experimental.pallas import tpu as pltpu
from jax.experimental.pallas import tpu_sc as plsc

Fin = 128
Fqk = 128
H = 16
Fh = Fqk // H
N = 10000
E = 320000

NW = 32
NP = 10240
BN = NP // NW
SUB = 2
HB = BN // SUB
NSB = NP // HB
RW = 224
S_MAX = NP // RW + 2
SP = 48
SP2 = 56
CH = 1024
EPAD = E + 2 * CH + 32
NG = EPAD // CH + 1
NGP = ((NG + 16 + 7) // 8) * 8
PR = 10752


def _proj_body(x_ref, w_ref, o_ref):
    o_ref[...] = lax.dot_general(
        x_ref[...], w_ref[...],
        dimension_numbers=(((1,), (1,)), ((), ())),
        preferred_element_type=jnp.float32,
    )


def _sc_body(proj_h, dst_h, wlo_h, split_h, rs_h, ncg_h, ncs_h, out_h,
             q_blk, ost, win, d_ch, wlo_v, spl_v, rs_v, ncg_v, ncs_v, sem):
    wid = lax.axis_index("s") * 2 + lax.axis_index("c")
    n0 = wid * BN
    pltpu.sync_copy(rs_h.at[pl.ds(n0, BN + 16)], rs_v)
    pltpu.sync_copy(ncg_h, ncg_v)

    zero = jnp.zeros((16,), jnp.float32)

    for sb in range(SUB):
        nb = n0 + sb * HB
        bid = wid * SUB + sb
        pltpu.sync_copy(proj_h.at[pl.ds(nb, HB), pl.ds(0, Fqk)], q_blk)
        pltpu.sync_copy(wlo_h.at[pl.ds(bid * SP, SP + 16)], wlo_v)
        pltpu.sync_copy(split_h.at[pl.ds(bid * SP2, SP2 + 16)], spl_v)
        pltpu.sync_copy(ncs_h.at[pl.ds(bid * SP2, SP2 + 16)], ncs_v)

        def _zero(ln, _):
            for f in range(Fh):
                ost[ln, pl.ds(f * H, H)] = zero
            return _

        lax.fori_loop(0, HB, _zero, None)

        def _rs(ln):
            return rs_v[pl.ds(sb * HB + ln, 16)][0]

        def _edges(ln, base, wl, a_e, b_e, st):
            lnq = jnp.minimum(ln, HB - 1)
            qf = [q_blk[lnq, pl.ds(f * H, H)] for f in range(Fh)]

            def _edge(e, st2):
                es, a = st2
                j = e - base
                l = d_ch[pl.ds(j, 16)][0] - wl
                aw = qf[0] * win[l, pl.ds(0, H)]
                for f in range(1, Fh):
                    aw = aw + qf[f] * win[l, pl.ds(f * H, H)]
                ex = jnp.exp(aw)
                a = [a[f] + ex * win[l, pl.ds(Fqk + f * H, H)]
                     for f in range(Fh)]
                return (es + ex, a)

            return plsc.parallel_loop(a_e, b_e, unroll=4, carry=st)(_edge)

        def _flush(ln, es, a):
            inv = jnp.where(es > 0.0, 1.0 / es, 0.0)
            for f in range(Fh):
                ost[ln, pl.ds(f * H, H)] = a[f] * inv

        def _window(k, st):
            sp = spl_v[pl.ds(k, 16)]
            lo = sp[0]
            hi = sp[1]
            wl = pl.multiple_of(wlo_v[pl.ds(k, 16)][0], 8)

            @pl.when(hi > lo)
            def _():
                pltpu.async_copy(
                    proj_h.at[pl.ds(wl, RW), pl.ds(Fqk, 2 * Fqk)],
                    win, sem).wait()

            ca0 = lo // CH
            nch = jnp.where(hi > lo, (hi - 1) // CH - ca0 + 1, 0)
            nc_end = ncs_v[pl.ds(k, 16)][1]

            def _chunk(c, st2):
                ca = ca0 + c
                base = ca * CH
                pltpu.sync_copy(dst_h.at[pl.ds(base, CH + 16)], d_ch)
                elo_c = jnp.maximum(lo, base)
                grid_end = base + CH
                ehi_c = jnp.minimum(hi, grid_end)

                nc_grid = ncg_v[pl.ds(ca + 1, 16)][0]
                nc_abs = jnp.where(grid_end <= hi, nc_grid, nc_end)
                ln_exit = jnp.clip(nc_abs - nb, 0, HB)

                ln0, es, a = st2

                def _node(ln, st3):
                    es2, a2 = st3
                    a_e = jnp.maximum(_rs(ln), elo_c)
                    es2, a2 = _edges(ln, base, wl, a_e, _rs(ln + 1),
                                     (es2, a2))
                    _flush(ln, es2, a2)
                    return (zero, [zero] * Fh)

                es, a = lax.fori_loop(ln0, ln_exit, _node, (es, a))
                a_e = jnp.maximum(_rs(ln_exit), elo_c)
                es, a = _edges(ln_exit, base, wl, a_e, ehi_c, (es, a))
                return (ln_exit, es, a)

            return lax.fori_loop(0, nch, _chunk, st)

        lax.fori_loop(0, S_MAX, _window, (jnp.int32(0), zero, [zero] * Fh))

        pltpu.sync_copy(ost, out_h.at[pl.ds(nb, HB)])


@functools.lru_cache(maxsize=None)
def _sc_attn():
    return pl.kernel(
        _sc_body,
        out_type=jax.ShapeDtypeStruct((NP, Fqk), jnp.float32),
        mesh=plsc.VectorSubcoreMesh(core_axis_name="c", subcore_axis_name="s"),
        scratch_types=[
            pltpu.VMEM((HB, Fqk), jnp.float32),
            pltpu.VMEM((HB, Fqk), jnp.float32),
            pltpu.VMEM((RW, 2 * Fqk), jnp.float32),
            pltpu.VMEM((CH + 16,), jnp.int32),
            pltpu.VMEM((SP + 16,), jnp.int32),
            pltpu.VMEM((SP2 + 16,), jnp.int32),
            pltpu.VMEM((BN + 16,), jnp.int32),
            pltpu.VMEM((NGP,), jnp.int32),
            pltpu.VMEM((SP2 + 16,), jnp.int32),
            pltpu.SemaphoreType.DMA,
        ],
    )


_PERM = np.array([[h * Fh + f for h in range(H)] for f in range(Fh)]).reshape(-1)


def kernel(x, batch, ei, W):
    del batch
    src = ei[0].astype(jnp.int32)
    dst = ei[1].astype(jnp.int32)

    scaling = float(Fh) ** (-0.5)
    Wq = W[:Fqk][_PERM] * scaling
    Wk = W[Fqk:2 * Fqk][_PERM]
    Wv = W[2 * Fqk:][_PERM]
    Wp = jnp.concatenate([Wq, Wk, Wv], axis=0)

    x_pad = jnp.zeros((PR, Fin), jnp.float32).at[:N].set(x)

    RB = 512
    proj = pl.pallas_call(
        _proj_body,
        grid=(PR // RB,),
        in_specs=[
            pl.BlockSpec((RB, Fin), lambda i: (i, 0)),
            pl.BlockSpec((3 * Fqk, Fin), lambda i: (0, 0)),
        ],
        out_specs=pl.BlockSpec((RB, 3 * Fqk), lambda i: (i, 0)),
        out_shape=jax.ShapeDtypeStruct((PR, 3 * Fqk), jnp.float32),
    )(x_pad, Wp)

    row_start = jnp.concatenate([
        jnp.zeros((1,), jnp.int32),
        jnp.cumsum(jnp.bincount(src, length=NP)).astype(jnp.int32)])
    rs_pad = jnp.concatenate(
        [row_start, jnp.full((15,), E, jnp.int32)])
    dd = jnp.concatenate([
        jnp.zeros((1,), jnp.int32),
        jnp.cumsum(jnp.bincount(dst, length=NP)).astype(jnp.int32)])
    e_lo_b = row_start[:NP:HB]
    e_hi_b = jnp.concatenate([e_lo_b[1:], jnp.full((1,), E, jnp.int32)])
    d0 = dst[jnp.minimum(e_lo_b, E - 1)]
    d0_8 = (d0 // 8) * 8

    ks = jnp.arange(S_MAX + 1, dtype=jnp.int32)
    targets = d0_8[:, None] + ks[None, :] * RW
    tclip = jnp.clip(targets, 0, NP)
    split = dd[tclip]
    split = jnp.clip(split, e_lo_b[:, None], e_hi_b[:, None])
    split_p = jnp.concatenate([
        jnp.zeros((NSB, SP2), jnp.int32).at[:, :S_MAX + 1].set(split)
        .reshape(-1),
        jnp.zeros((16,), jnp.int32)])
    wlo = jnp.minimum(targets[:, :S_MAX], NP)
    wlo_p = jnp.concatenate([
        jnp.zeros((NSB, SP), jnp.int32).at[:, :S_MAX].set(wlo).reshape(-1),
        jnp.zeros((16,), jnp.int32)])

    dst_p = jnp.zeros((EPAD,), jnp.int32).at[:E].set(dst)

    def _nc(x):
        return jnp.where(x >= E, NP,
                         src[jnp.minimum(x, E - 1)]).astype(jnp.int32)

    ncg = _nc(jnp.arange(NG, dtype=jnp.int32) * CH)
    ncg_p = jnp.concatenate(
        [ncg, jnp.zeros((NGP - NG,), jnp.int32)])
    ncs = _nc(split)
    ncs_p = jnp.concatenate([
        jnp.zeros((NSB, SP2), jnp.int32).at[:, :S_MAX + 1].set(ncs)
        .reshape(-1),
        jnp.zeros((16,), jnp.int32)])

    out = _sc_attn()(proj, dst_p, wlo_p, split_p, rs_pad, ncg_p, ncs_p)
    return out[:N].reshape(N, Fh, H).transpose(0, 2, 1).reshape(N, H * Fh)


# speedup vs baseline: 201.1719x; 1.0493x over previous
import functools

import jax
import jax.numpy as jnp
import numpy as np
from jax import lax
from jax.experimental import pallas as pl
from jax.experimental.pallas import tpu as pltpu
from jax.experimental.pallas import tpu_sc as plsc

Fin = 128
Fqk = 128
H = 16
Fh = Fqk // H
N = 10000
E = 320000

NW = 32
NP = 10240
BN = NP // NW
SUB = 2
HB = BN // SUB
NSB = NP // HB
RW = 224
S_MAX = -(-NP // RW)
SP = 64
CH = 2048
EPAD = E + 2 * CH + 32
NG = EPAD // CH + 1
NGP = ((NG + 16 + 7) // 8) * 8
PR = 10752


def _proj_body(x_ref, w_ref, o_ref):
    o_ref[...] = lax.dot_general(
        x_ref[...], w_ref[...],
        dimension_numbers=(((1,), (1,)), ((), ())),
        preferred_element_type=jnp.float32,
    )


def _sc_body(proj_h, dst_h, tab_h, rs_h, out_h,
             q_blk, ost, win, d_ch, tab_v, rs_v, sem):
    wid = lax.axis_index("s") * 2 + lax.axis_index("c")
    n0 = wid * BN
    pltpu.sync_copy(rs_h.at[pl.ds(n0, BN + 16)], rs_v)
    pltpu.sync_copy(tab_h, tab_v)

    def _spl(i):
        return tab_v[pl.ds(i, 16)]

    def _ncs(i):
        return tab_v[pl.ds(SP + i, 16)]

    def _ncg(i):
        return tab_v[pl.ds(2 * SP + i, 16)]

    zero = jnp.zeros((16,), jnp.float32)

    def _rs0(i):
        return rs_v[pl.ds(i, 16)][0]

    for sb in range(SUB):
        nb = n0 + sb * HB
        pltpu.sync_copy(proj_h.at[pl.ds(nb, HB), pl.ds(0, Fqk)], q_blk)
        e_lo_b = _rs0(sb * HB)
        e_hi_b = _rs0(sb * HB + HB)

        def _zero(ln, _):
            for f in range(Fh):
                ost[ln, pl.ds(f * H, H)] = zero
            return _

        lax.fori_loop(0, HB, _zero, None)

        def _rs(ln):
            return rs_v[pl.ds(sb * HB + ln, 16)][0]

        def _edges(ln, base, wl, a_e, b_e, st):
            lnq = jnp.minimum(ln, HB - 1)
            qf = [q_blk[lnq, pl.ds(f * H, H)] for f in range(Fh)]

            def _edge(e, st2):
                es, a = st2
                j = e - base
                l = d_ch[pl.ds(j, 16)][0] - wl
                aw = qf[0] * win[l, pl.ds(0, H)]
                for f in range(1, Fh):
                    aw = aw + qf[f] * win[l, pl.ds(f * H, H)]
                ex = jnp.exp(aw)
                a = [a[f] + ex * win[l, pl.ds(Fqk + f * H, H)]
                     for f in range(Fh)]
                return (es + ex, a)

            return plsc.parallel_loop(a_e, b_e, unroll=4, carry=st)(_edge)

        def _flush(ln, es, a):
            inv = jnp.where(es > 0.0, 1.0 / es, 0.0)
            for f in range(Fh):
                ost[ln, pl.ds(f * H, H)] = a[f] * inv

        def _window(k, st):
            sp = _spl(k)
            lo = jnp.maximum(sp[0], e_lo_b)
            hi = jnp.minimum(sp[1], e_hi_b)
            wl = pl.multiple_of(k * RW, 8)

            @pl.when(hi > lo)
            def _():
                pltpu.async_copy(
                    proj_h.at[pl.ds(wl, RW), pl.ds(Fqk, 2 * Fqk)],
                    win, sem).wait()

            ca0 = lo // CH
            nch = jnp.where(hi > lo, (hi - 1) // CH - ca0 + 1, 0)
            nc_end = _ncs(k)[1]


            def _chunk(c, st2):
                ca = ca0 + c
                base = ca * CH
                pltpu.sync_copy(dst_h.at[pl.ds(base, CH + 16)], d_ch)
                elo_c = jnp.maximum(lo, base)
                grid_end = base + CH
                ehi_c = jnp.minimum(hi, grid_end)

                nc_grid = _ncg(ca + 1)[0]
                nc_abs = jnp.where(grid_end <= hi, nc_grid, nc_end)
                ln_exit = jnp.clip(nc_abs - nb, 0, HB)

                ln0, es, a = st2

                def _node(ln, st3):
                    es2, a2 = st3
                    a_e = jnp.maximum(_rs(ln), elo_c)
                    es2, a2 = _edges(ln, base, wl, a_e, _rs(ln + 1),
                                     (es2, a2))
                    _flush(ln, es2, a2)
                    return (zero, [zero] * Fh)

                es, a = lax.fori_loop(ln0, ln_exit, _node, (es, a))
                a_e = jnp.maximum(_rs(ln_exit), elo_c)
                es, a = _edges(ln_exit, base, wl, a_e, ehi_c, (es, a))
                return (ln_exit, es, a)

            return lax.fori_loop(0, nch, _chunk, st)

        lax.fori_loop(0, S_MAX, _window, (jnp.int32(0), zero, [zero] * Fh))

        pltpu.sync_copy(ost, out_h.at[pl.ds(nb, HB)])


@functools.lru_cache(maxsize=None)
def _sc_attn():
    return pl.kernel(
        _sc_body,
        out_type=jax.ShapeDtypeStruct((NP, Fqk), jnp.float32),
        mesh=plsc.VectorSubcoreMesh(core_axis_name="c", subcore_axis_name="s"),
        scratch_types=[
            pltpu.VMEM((HB, Fqk), jnp.float32),
            pltpu.VMEM((HB, Fqk), jnp.float32),
            pltpu.VMEM((RW, 2 * Fqk), jnp.float32),
            pltpu.VMEM((CH + 16,), jnp.int32),
            pltpu.VMEM((2 * SP + NGP,), jnp.int32),
            pltpu.VMEM((BN + 16,), jnp.int32),
            pltpu.SemaphoreType.DMA,
        ],
    )


_PERM = np.array([[h * Fh + f for h in range(H)] for f in range(Fh)]).reshape(-1)


def kernel(x, batch, ei, W):
    del batch
    src = ei[0].astype(jnp.int32)
    dst = ei[1].astype(jnp.int32)

    scaling = float(Fh) ** (-0.5)
    Wq = W[:Fqk][_PERM] * scaling
    Wk = W[Fqk:2 * Fqk][_PERM]
    Wv = W[2 * Fqk:][_PERM]
    Wp = jnp.concatenate([Wq, Wk, Wv], axis=0)

    x_pad = jnp.zeros((PR, Fin), jnp.float32).at[:N].set(x)

    RB = 512
    proj = pl.pallas_call(
        _proj_body,
        grid=(PR // RB,),
        in_specs=[
            pl.BlockSpec((RB, Fin), lambda i: (i, 0)),
            pl.BlockSpec((3 * Fqk, Fin), lambda i: (0, 0)),
        ],
        out_specs=pl.BlockSpec((RB, 3 * Fqk), lambda i: (i, 0)),
        out_shape=jax.ShapeDtypeStruct((PR, 3 * Fqk), jnp.float32),
    )(x_pad, Wp)

    cnts = jnp.bincount(jnp.concatenate([src, dst + NP]), length=2 * NP)
    row_start = jnp.concatenate([
        jnp.zeros((1,), jnp.int32),
        jnp.cumsum(cnts[:NP]).astype(jnp.int32)])
    rs_pad = jnp.concatenate(
        [row_start, jnp.full((15,), E, jnp.int32)])
    dd = jnp.concatenate([
        jnp.zeros((1,), jnp.int32),
        jnp.cumsum(cnts[NP:]).astype(jnp.int32),
        jnp.full((S_MAX * RW - NP,), E, jnp.int32)])
    splitg = dd[::RW]

    dst_p = jnp.zeros((EPAD,), jnp.int32).at[:E].set(dst)

    def _nc(x):
        return jnp.where(x >= E, NP,
                         src[jnp.minimum(x, E - 1)]).astype(jnp.int32)

    s_str = src[::CH].astype(jnp.int32)
    ncsg = _nc(splitg)
    zpad = jnp.zeros((SP - S_MAX - 1,), jnp.int32)
    tab = jnp.concatenate([
        splitg, zpad,
        ncsg, zpad,
        s_str, jnp.full((NGP - s_str.shape[0],), NP, jnp.int32)])

    out = _sc_attn()(proj, dst_p, tab, rs_pad)
    return out[:N].reshape(N, Fh, H).transpose(0, 2, 1).reshape(N, H * Fh)
